# Initial kernel scaffold; baseline (speedup 1.0000x reference)
import jax
import jax.numpy as jnp
from jax.experimental import pallas as pl


def kernel(x_t, timestep, edge_index, y, W1, b1, ln1_g, ln1_b, W2, b2, ln2_g, ln2_b, W3, b3, ln3_g, ln3_b):
    raise NotImplementedError("write your pallas kernel here")


# speedup vs baseline: 14.1502x; 14.1502x over previous
import functools
import math

import jax
import jax.numpy as jnp
from jax import lax
from jax.experimental import pallas as pl
from jax.experimental.pallas import tpu as pltpu
from jax.experimental.pallas import tpu_sc as plsc

NS = 16
NC = 2
CW = 128
CB = 8
NBLK = 20
EPTP = NBLK * CB * CW
NACC = 10112
SPARE = 10104
ZPT = NACC // NS
WPT = 632
OPT = 640
NPOUT = OPT * NS


def _make_spmm():
  mesh = plsc.VectorSubcoreMesh(core_axis_name="c", subcore_axis_name="s")

  @functools.partial(
      pl.kernel,
      out_type=(
          jax.ShapeDtypeStruct((NPOUT, 128), jnp.float32),
          jax.ShapeDtypeStruct((NPOUT, 128), jnp.float32),
      ),
      mesh=mesh,
      scratch_types=[
          pltpu.VMEM_SHARED((NACC, 128), jnp.float32),
          pltpu.VMEM((CB, CW), jnp.int32),
          pltpu.VMEM((CB, CW), jnp.int32),
          pltpu.VMEM((CW, 128), jnp.float32),
          pltpu.SemaphoreType.DMA,
      ],
  )
  def spmm(src0_hbm, src1_hbm, gidx_hbm, sidx_hbm, zeros_hbm,
           out0_hbm, out1_hbm, acc, gi_b, si_b, rows_v, sem):
    cid = lax.axis_index("c")
    sid = lax.axis_index("s")
    pltpu.sync_copy(zeros_hbm, acc.at[pl.ds(sid * ZPT, ZPT)])
    plsc.subcore_barrier()

    def body(k, carry):
      pltpu.sync_copy(gidx_hbm.at[sid, k], gi_b)
      pltpu.sync_copy(sidx_hbm.at[sid, k], si_b)
      for q in range(CB):
        @pl.when(cid == 0)
        def _():
          pltpu.async_copy(src0_hbm.at[gi_b.at[q]], rows_v, sem).wait()

        @pl.when(cid == 1)
        def _():
          pltpu.async_copy(src1_hbm.at[gi_b.at[q]], rows_v, sem).wait()

        pltpu.sync_copy(rows_v, acc.at[si_b.at[q]], add=True)
      return carry

    lax.fori_loop(0, NBLK, body, 0)
    plsc.subcore_barrier()

    @pl.when(cid == 0)
    def _():
      pltpu.sync_copy(acc.at[pl.ds(sid * WPT, WPT)],
                      out0_hbm.at[pl.ds(sid * OPT, WPT)])

    @pl.when(cid == 1)
    def _():
      pltpu.sync_copy(acc.at[pl.ds(sid * WPT, WPT)],
                      out1_hbm.at[pl.ds(sid * OPT, WPT)])

  return spmm


def _make_degrees():
  mesh = plsc.VectorSubcoreMesh(core_axis_name="c", subcore_axis_name="s")

  @functools.partial(
      pl.kernel,
      out_type=(
          jax.ShapeDtypeStruct((NPOUT, 128), jnp.float32),
          jax.ShapeDtypeStruct((NPOUT, 128), jnp.float32),
      ),
      mesh=mesh,
      scratch_types=[
          pltpu.VMEM_SHARED((NACC, 128), jnp.float32),
          pltpu.VMEM((CB, CW), jnp.int32),
          pltpu.VMEM((CW, 128), jnp.float32),
      ],
  )
  def degrees(sidx_hbm, ones_hbm, zeros_hbm, dx_hbm, dy_hbm,
              acc, si_b, ones_v):
    cid = lax.axis_index("c")
    sid = lax.axis_index("s")
    pltpu.sync_copy(zeros_hbm, acc.at[pl.ds(sid * ZPT, ZPT)])
    pltpu.sync_copy(ones_hbm, ones_v)
    plsc.subcore_barrier()

    def body(k, carry):
      pltpu.sync_copy(sidx_hbm.at[cid, sid, k], si_b)
      for q in range(CB):
        pltpu.sync_copy(ones_v, acc.at[si_b.at[q]], add=True)
      return carry

    lax.fori_loop(0, NBLK, body, 0)
    plsc.subcore_barrier()

    @pl.when(cid == 0)
    def _():
      pltpu.sync_copy(acc.at[pl.ds(sid * WPT, WPT)],
                      dx_hbm.at[pl.ds(sid * OPT, WPT)])

    @pl.when(cid == 1)
    def _():
      pltpu.sync_copy(acc.at[pl.ds(sid * WPT, WPT)],
                      dy_hbm.at[pl.ds(sid * OPT, WPT)])

  return degrees


def _layernorm(z, g, b):
  m = jnp.mean(z, axis=1, keepdims=True)
  v = jnp.mean((z - m) ** 2, axis=1, keepdims=True)
  return (z - m) * lax.rsqrt(v + 1e-5) * g + b


def _deg_inv_sqrt(deg_ref):
  deg = deg_ref[...][:, 0:1]
  return jnp.where(deg > 0, lax.rsqrt(deg), 0.0)


def _enc_x_body(xt, tsb, degx, w1a, w1b, b1, g1, bb1, fr2,
                zx0_o, zx1_o, p0_o, p1_o):
  ang = tsb[...] * fr2[...]
  lane = lax.broadcasted_iota(jnp.int32, ang.shape, 1)
  te = jnp.where(lane < 64, jnp.cos(ang), jnp.sin(ang))
  z = jnp.dot(xt[...], w1a[...], preferred_element_type=jnp.float32)
  z = z + jnp.dot(te, w1b[...], preferred_element_type=jnp.float32)
  z = _layernorm(z + b1[...], g1[...], bb1[...])
  z = jnp.maximum(z, 0.0)
  p = z * _deg_inv_sqrt(degx)
  zx0_o[...] = z[:, :128]
  zx1_o[...] = z[:, 128:]
  p0_o[...] = p[:, :128]
  p1_o[...] = p[:, 128:]


def _enc_y_body(yy, degy, w2, b2, g2, bb2, q0_o, q1_o):
  z = jnp.dot(yy[...], w2[...], preferred_element_type=jnp.float32)
  z = _layernorm(z + b2[...], g2[...], bb2[...])
  z = jnp.maximum(z, 0.0)
  q = z * _deg_inv_sqrt(degy)
  q0_o[...] = q[:, :128]
  q1_o[...] = q[:, 128:]


def _scale_body(p0, p1, s0, s1, deg_r, o0, o1):
  deg = deg_r[...][:, 0:1]
  d2 = jnp.where(deg > 0, 1.0 / deg, 0.0)
  o0[...] = p0[...] + d2 * s0[...]
  o1[...] = p1[...] + d2 * s1[...]


def _dec_body(zx0, zx1, v0, v1, degx, w3a, w3b, b3, g3, bb3, out_o):
  dx = _deg_inv_sqrt(degx)
  h0 = (zx0[...] + dx * v0[...]) * 0.25
  h1 = (zx1[...] + dx * v1[...]) * 0.25
  o = jnp.dot(h0, w3a[...], preferred_element_type=jnp.float32)
  o = o + jnp.dot(h1, w3b[...], preferred_element_type=jnp.float32)
  o = _layernorm(o + b3[...], g3[...], bb3[...])
  out_o[...] = jnp.tanh(o)


def _row_spec(r, cols):
  return pl.BlockSpec((r, cols), lambda i: (i, 0))


def _full_spec(shape):
  return pl.BlockSpec(shape, lambda i: tuple(0 for _ in shape))


def kernel(x_t, timestep, edge_index, y, W1, b1, ln1_g, ln1_b,
           W2, b2, ln2_g, ln2_b, W3, b3, ln3_g, ln3_b):
  N, D = x_t.shape
  M = y.shape[0]
  E = edge_index.shape[1]
  H = W1.shape[0]
  O = W3.shape[0]
  ept = E // NS

  f32 = jnp.float32

  def pack_idx(ei, padval):
    t = ei.astype(jnp.int32).reshape(NS, ept)
    t = jnp.pad(t, ((0, 0), (0, EPTP - ept)), constant_values=padval)
    return t.reshape(NS, NBLK, CB, CW)

  e0 = edge_index[0]
  e1 = edge_index[1]
  e0g = pack_idx(e0, 0)
  e0s = pack_idx(e0, SPARE)
  e1g = pack_idx(e1, 0)
  e1s = pack_idx(e1, SPARE)
  sidx2 = jnp.stack([e0s, e1s])

  def unpack(o):
    return o.reshape(NS, OPT, 128)[:, :WPT].reshape(NS * WPT, 128)[:N]

  half = D // 2
  freqs = jnp.exp(-math.log(10000.0) *
                  jnp.arange(0, half, dtype=f32) / half)
  fr2 = jnp.concatenate([freqs, freqs]).reshape(1, D)
  tsb = jnp.broadcast_to(timestep.astype(f32)[:, None], (N, D))

  zerosb = jnp.zeros((ZPT, 128), f32)
  onesb = jnp.ones((CW, 128), f32)

  degrees = _make_degrees()
  spmm_p = _make_spmm()

  def spmm(a0, a1, gi, si):
    o0, o1 = spmm_p(a0, a1, gi, si, zerosb)
    return unpack(o0), unpack(o1)

  degxf, degyf = degrees(sidx2, onesb, zerosb)
  degx = unpack(degxf)
  degy = unpack(degyf)

  R = 1000
  grid = (N // R,)
  w1aT = W1[:, :D].T
  w1bT = W1[:, D:].T
  zx0, zx1, p0, p1 = pl.pallas_call(
      _enc_x_body,
      grid=grid,
      in_specs=[
          _row_spec(R, D), _row_spec(R, D), _row_spec(R, 128),
          _full_spec((D, H)), _full_spec((D, H)), _full_spec((1, H)),
          _full_spec((1, H)), _full_spec((1, H)), _full_spec((1, D)),
      ],
      out_specs=[_row_spec(R, 128)] * 4,
      out_shape=[jax.ShapeDtypeStruct((N, 128), f32)] * 4,
  )(x_t, tsb, degx, w1aT, w1bT, b1.reshape(1, H), ln1_g.reshape(1, H),
    ln1_b.reshape(1, H), fr2)

  q0, q1 = pl.pallas_call(
      _enc_y_body,
      grid=grid,
      in_specs=[
          _row_spec(R, D), _row_spec(R, 128),
          _full_spec((D, H)), _full_spec((1, H)), _full_spec((1, H)),
          _full_spec((1, H)),
      ],
      out_specs=[_row_spec(R, 128)] * 2,
      out_shape=[jax.ShapeDtypeStruct((M, 128), f32)] * 2,
  )(y, degy, W2.T, b2.reshape(1, H), ln2_g.reshape(1, H),
    ln2_b.reshape(1, H))

  def scale(a0, a1, s0, s1, degs):
    return pl.pallas_call(
        _scale_body,
        grid=grid,
        in_specs=[_row_spec(R, 128)] * 4 + [_row_spec(R, 128)],
        out_specs=[_row_spec(R, 128)] * 2,
        out_shape=[jax.ShapeDtypeStruct((N, 128), f32)] * 2,
    )(a0, a1, s0, s1, degs)

  s0, s1 = spmm(q0, q1, e1g, e0s)
  c0, c1 = scale(p0, p1, s0, s1, degx)
  u0, u1 = spmm(c0, c1, e0g, e1s)
  d0, d1 = scale(q0, q1, u0, u1, degy)
  v0, v1 = spmm(d0, d1, e1g, e0s)

  w3T = W3.T
  out = pl.pallas_call(
      _dec_body,
      grid=grid,
      in_specs=[_row_spec(R, 128)] * 4 + [
          _row_spec(R, 128),
          _full_spec((D, O)), _full_spec((D, O)), _full_spec((1, O)),
          _full_spec((1, O)), _full_spec((1, O)),
      ],
      out_specs=_row_spec(R, O),
      out_shape=jax.ShapeDtypeStruct((N, O), f32),
  )(zx0, zx1, v0, v1, degx, w3T[:D], w3T[D:], b3.reshape(1, O),
    ln3_g.reshape(1, O), ln3_b.reshape(1, O))
  return out


# speedup vs baseline: 15.3963x; 1.0881x over previous
import functools
import math

import jax
import jax.numpy as jnp
from jax import lax
from jax.experimental import pallas as pl
from jax.experimental.pallas import tpu as pltpu
from jax.experimental.pallas import tpu_sc as plsc

NS = 16
NC = 2
CW = 128
CB = 8
NBLK = 20
EPTP = NBLK * CB * CW
NACC = 10112
SPARE = 10104
ZPT = NACC // NS
WPT = 632
OPT = 640
NPOUT = OPT * NS


def _make_spmm():
  mesh = plsc.VectorSubcoreMesh(core_axis_name="c", subcore_axis_name="s")

  @functools.partial(
      pl.kernel,
      out_type=jax.ShapeDtypeStruct((NC, NPOUT, 128), jnp.float32),
      mesh=mesh,
      scratch_types=[
          pltpu.VMEM_SHARED((NACC, 128), jnp.float32),
          pltpu.VMEM((CB, CW), jnp.int32),
          pltpu.VMEM((CB, CW), jnp.int32),
          pltpu.VMEM((CW, 128), jnp.float32),
          pltpu.VMEM((CW, 128), jnp.float32),
          pltpu.SemaphoreType.DMA,
          pltpu.SemaphoreType.DMA,
          pltpu.SemaphoreType.DMA,
      ],
  )
  def spmm(src_hbm, gidx_hbm, sidx_hbm, zeros_hbm, out_hbm,
           acc, gi_b, si_b, rows0, rows1, sg0, sg1, ss0):
    cid = lax.axis_index("c")
    sid = lax.axis_index("s")
    pltpu.sync_copy(zeros_hbm, acc.at[pl.ds(sid * ZPT, ZPT)])
    plsc.subcore_barrier()

    def block(k, carry):
      pltpu.sync_copy(gidx_hbm.at[cid, sid, k], gi_b)
      pltpu.sync_copy(sidx_hbm.at[cid, sid, k], si_b)
      for j in range(CB // 2):
        g0 = pltpu.async_copy(src_hbm.at[gi_b.at[2 * j]], rows0, sg0)
        g1 = pltpu.async_copy(src_hbm.at[gi_b.at[2 * j + 1]], rows1, sg1)
        g0.wait()
        g1.wait()
        pltpu.async_copy(rows0, acc.at[si_b.at[2 * j]], ss0,
                         add=True).wait()
        pltpu.async_copy(rows1, acc.at[si_b.at[2 * j + 1]], ss0,
                         add=True).wait()
      return carry

    lax.fori_loop(0, NBLK, block, 0)
    plsc.subcore_barrier()
    pltpu.sync_copy(acc.at[pl.ds(sid * WPT, WPT)],
                    out_hbm.at[cid, pl.ds(sid * OPT, WPT)])

  return spmm


def _make_degrees():
  mesh = plsc.VectorSubcoreMesh(core_axis_name="c", subcore_axis_name="s")

  @functools.partial(
      pl.kernel,
      out_type=jax.ShapeDtypeStruct((NC, NPOUT, 16), jnp.float32),
      mesh=mesh,
      scratch_types=[
          pltpu.VMEM_SHARED((NACC, 16), jnp.float32),
          pltpu.VMEM((CB, CW), jnp.int32),
          pltpu.VMEM((CW, 16), jnp.float32),
          pltpu.SemaphoreType.DMA,
      ],
  )
  def degrees(sidx_hbm, ones_hbm, zeros_hbm, deg_hbm, acc, si_b, ones_v, ssem):
    cid = lax.axis_index("c")
    sid = lax.axis_index("s")
    pltpu.sync_copy(zeros_hbm, acc.at[pl.ds(sid * ZPT, ZPT)])
    pltpu.sync_copy(ones_hbm, ones_v)
    plsc.subcore_barrier()

    def block(k, carry):
      pltpu.sync_copy(sidx_hbm.at[cid, sid, k], si_b)
      for q in range(CB):
        pltpu.async_copy(ones_v, acc.at[si_b.at[q]], ssem, add=True).wait()
      return carry

    lax.fori_loop(0, NBLK, block, 0)
    plsc.subcore_barrier()
    pltpu.sync_copy(acc.at[pl.ds(sid * WPT, WPT)],
                    deg_hbm.at[cid, pl.ds(sid * OPT, WPT)])

  return degrees


def _layernorm(z, g, b):
  m = jnp.mean(z, axis=1, keepdims=True)
  v = jnp.mean((z - m) ** 2, axis=1, keepdims=True)
  return (z - m) * lax.rsqrt(v + 1e-5) * g + b


def _deg_inv_sqrt(deg_ref):
  deg = deg_ref[...][:, 0:1]
  return jnp.where(deg > 0, lax.rsqrt(deg), 0.0)


def _enc_x_body(xt, tsb, degx, w1a, w1b, b1, g1, bb1, fr2,
                zx0_o, zx1_o, p0_o, p1_o):
  ang = tsb[...] * fr2[...]
  lane = lax.broadcasted_iota(jnp.int32, ang.shape, 1)
  te = jnp.where(lane < 64, jnp.cos(ang), jnp.sin(ang))
  z = jnp.dot(xt[...], w1a[...], preferred_element_type=jnp.float32)
  z = z + jnp.dot(te, w1b[...], preferred_element_type=jnp.float32)
  z = _layernorm(z + b1[...], g1[...], bb1[...])
  z = jnp.maximum(z, 0.0)
  p = z * _deg_inv_sqrt(degx)
  zx0_o[...] = z[:, :128]
  zx1_o[...] = z[:, 128:]
  p0_o[...] = p[:, :128]
  p1_o[...] = p[:, 128:]


def _enc_y_body(yy, degy, w2, b2, g2, bb2, q0_o, q1_o):
  z = jnp.dot(yy[...], w2[...], preferred_element_type=jnp.float32)
  z = _layernorm(z + b2[...], g2[...], bb2[...])
  z = jnp.maximum(z, 0.0)
  q = z * _deg_inv_sqrt(degy)
  q0_o[...] = q[:, :128]
  q1_o[...] = q[:, 128:]


def _scale_body(p0, p1, s0, s1, deg_r, o0, o1):
  deg = deg_r[...][:, 0:1]
  d2 = jnp.where(deg > 0, 1.0 / deg, 0.0)
  o0[...] = p0[...] + d2 * s0[...]
  o1[...] = p1[...] + d2 * s1[...]


def _dec_body(zx0, zx1, v0, v1, degx, w3a, w3b, b3, g3, bb3, out_o):
  dx = _deg_inv_sqrt(degx)
  h0 = (zx0[...] + dx * v0[...]) * 0.25
  h1 = (zx1[...] + dx * v1[...]) * 0.25
  o = jnp.dot(h0, w3a[...], preferred_element_type=jnp.float32)
  o = o + jnp.dot(h1, w3b[...], preferred_element_type=jnp.float32)
  o = _layernorm(o + b3[...], g3[...], bb3[...])
  out_o[...] = jnp.tanh(o)


def _row_spec(r, cols):
  return pl.BlockSpec((r, cols), lambda i: (i, 0))


def _full_spec(shape):
  return pl.BlockSpec(shape, lambda i: tuple(0 for _ in shape))


def kernel(x_t, timestep, edge_index, y, W1, b1, ln1_g, ln1_b,
           W2, b2, ln2_g, ln2_b, W3, b3, ln3_g, ln3_b):
  N, D = x_t.shape
  M = y.shape[0]
  E = edge_index.shape[1]
  H = W1.shape[0]
  O = W3.shape[0]
  ept = E // NS

  f32 = jnp.float32

  def pack_idx(ei, padval):
    t = ei.astype(jnp.int32).reshape(NS, ept)
    t = jnp.pad(t, ((0, 0), (0, EPTP - ept)), constant_values=padval)
    return t.reshape(NS, NBLK, CB, CW)

  e0 = edge_index[0]
  e1 = edge_index[1]
  e0g = pack_idx(e0, 0)
  e0s = pack_idx(e0, SPARE)
  e1g = pack_idx(e1, 0)
  e1s = pack_idx(e1, SPARE)
  ge0 = jnp.stack([e0g, e0g + N])
  ge1 = jnp.stack([e1g, e1g + M])
  se0 = jnp.stack([e0s, e0s])
  se1 = jnp.stack([e1s, e1s])
  sidx2 = jnp.stack([e0s, e1s])

  def unpack(o):
    cols = o.shape[-1]
    return o.reshape(NS, OPT, cols)[:, :WPT].reshape(NS * WPT, cols)[:N]

  half = D // 2
  freqs = jnp.exp(-math.log(10000.0) *
                  jnp.arange(0, half, dtype=f32) / half)
  fr2 = jnp.concatenate([freqs, freqs]).reshape(1, D)
  tsb = jnp.broadcast_to(timestep.astype(f32)[:, None], (N, D))

  zerosb = jnp.zeros((ZPT, 128), f32)
  zeros16 = jnp.zeros((ZPT, 16), f32)
  ones16 = jnp.ones((CW, 16), f32)

  degrees = _make_degrees()
  spmm_p = _make_spmm()

  def spmm(a0, a1, gi, si):
    o = spmm_p(jnp.concatenate([a0, a1], axis=0), gi, si, zerosb)
    return unpack(o[0]), unpack(o[1])

  degf = degrees(sidx2, ones16, zeros16)
  degx = unpack(degf[0])
  degy = unpack(degf[1])

  R = 1000
  grid = (N // R,)
  w1aT = W1[:, :D].T
  w1bT = W1[:, D:].T
  zx0, zx1, p0, p1 = pl.pallas_call(
      _enc_x_body,
      grid=grid,
      in_specs=[
          _row_spec(R, D), _row_spec(R, D), _row_spec(R, 16),
          _full_spec((D, H)), _full_spec((D, H)), _full_spec((1, H)),
          _full_spec((1, H)), _full_spec((1, H)), _full_spec((1, D)),
      ],
      out_specs=[_row_spec(R, 128)] * 4,
      out_shape=[jax.ShapeDtypeStruct((N, 128), f32)] * 4,
  )(x_t, tsb, degx, w1aT, w1bT, b1.reshape(1, H), ln1_g.reshape(1, H),
    ln1_b.reshape(1, H), fr2)

  q0, q1 = pl.pallas_call(
      _enc_y_body,
      grid=grid,
      in_specs=[
          _row_spec(R, D), _row_spec(R, 16),
          _full_spec((D, H)), _full_spec((1, H)), _full_spec((1, H)),
          _full_spec((1, H)),
      ],
      out_specs=[_row_spec(R, 128)] * 2,
      out_shape=[jax.ShapeDtypeStruct((M, 128), f32)] * 2,
  )(y, degy, W2.T, b2.reshape(1, H), ln2_g.reshape(1, H),
    ln2_b.reshape(1, H))

  def scale(a0, a1, s0, s1, degs):
    return pl.pallas_call(
        _scale_body,
        grid=grid,
        in_specs=[_row_spec(R, 128)] * 4 + [_row_spec(R, 16)],
        out_specs=[_row_spec(R, 128)] * 2,
        out_shape=[jax.ShapeDtypeStruct((N, 128), f32)] * 2,
    )(a0, a1, s0, s1, degs)

  s0, s1 = spmm(q0, q1, ge1, se0)
  c0, c1 = scale(p0, p1, s0, s1, degx)
  u0, u1 = spmm(c0, c1, ge0, se1)
  d0, d1 = scale(q0, q1, u0, u1, degy)
  v0, v1 = spmm(d0, d1, ge1, se0)

  w3T = W3.T
  out = pl.pallas_call(
      _dec_body,
      grid=grid,
      in_specs=[_row_spec(R, 128)] * 4 + [
          _row_spec(R, 16),
          _full_spec((D, O)), _full_spec((D, O)), _full_spec((1, O)),
          _full_spec((1, O)), _full_spec((1, O)),
      ],
      out_specs=_row_spec(R, O),
      out_shape=jax.ShapeDtypeStruct((N, O), f32),
  )(zx0, zx1, v0, v1, degx, w3T[:D], w3T[D:], b3.reshape(1, O),
    ln3_g.reshape(1, O), ln3_b.reshape(1, O))
  return out


# speedup vs baseline: 15.4014x; 1.0003x over previous
import functools
import math

import jax
import jax.numpy as jnp
from jax import lax
from jax.experimental import pallas as pl
from jax.experimental.pallas import tpu as pltpu
from jax.experimental.pallas import tpu_sc as plsc

NS = 16
NC = 2
CW = 128
CB = 8
NBLK = 20
EPTP = NBLK * CB * CW
NACC = 10112
SPARE = 10104
ZPT = NACC // NS
WPT = 632
OPT = 640
NPOUT = OPT * NS


def _make_spmm():
  mesh = plsc.VectorSubcoreMesh(core_axis_name="c", subcore_axis_name="s")

  @functools.partial(
      pl.kernel,
      out_type=jax.ShapeDtypeStruct((NC, NPOUT, 128), jnp.float32),
      mesh=mesh,
      scratch_types=[
          pltpu.VMEM_SHARED((NACC, 128), jnp.float32),
          pltpu.VMEM((CB, CW), jnp.int32),
          pltpu.VMEM((CB, CW), jnp.int32),
          pltpu.VMEM((CW, 128), jnp.float32),
          pltpu.VMEM((CW, 128), jnp.float32),
          pltpu.SemaphoreType.DMA,
          pltpu.SemaphoreType.DMA,
          pltpu.SemaphoreType.DMA,
          pltpu.SemaphoreType.DMA,
      ],
  )
  def spmm(src_hbm, gidx_hbm, sidx_hbm, zeros_hbm, out_hbm,
           acc, gi_b, si_b, rows0, rows1, sg0, sg1, ss0, ss1):
    cid = lax.axis_index("c")
    sid = lax.axis_index("s")
    pltpu.sync_copy(zeros_hbm, acc.at[pl.ds(sid * ZPT, ZPT)])
    plsc.subcore_barrier()

    def block(k, carry):
      pltpu.sync_copy(gidx_hbm.at[cid, sid, k], gi_b)
      pltpu.sync_copy(sidx_hbm.at[cid, sid, k], si_b)
      for j in range(CB // 2):
        g0 = pltpu.async_copy(src_hbm.at[gi_b.at[2 * j]], rows0, sg0)
        g1 = pltpu.async_copy(src_hbm.at[gi_b.at[2 * j + 1]], rows1, sg1)
        g0.wait()
        g1.wait()
        pltpu.async_copy(rows0, acc.at[si_b.at[2 * j]], ss0,
                         add=True).wait()
        pltpu.async_copy(rows1, acc.at[si_b.at[2 * j + 1]], ss0,
                         add=True).wait()
      return carry

    lax.fori_loop(0, NBLK, block, 0)
    plsc.subcore_barrier()
    pltpu.sync_copy(acc.at[pl.ds(sid * WPT, WPT)],
                    out_hbm.at[cid, pl.ds(sid * OPT, WPT)])

  return spmm


def _make_degrees():
  mesh = plsc.VectorSubcoreMesh(core_axis_name="c", subcore_axis_name="s")

  @functools.partial(
      pl.kernel,
      out_type=jax.ShapeDtypeStruct((NC, NPOUT, 16), jnp.float32),
      mesh=mesh,
      scratch_types=[
          pltpu.VMEM_SHARED((NACC, 16), jnp.float32),
          pltpu.VMEM((CB, CW), jnp.int32),
          pltpu.VMEM((CW, 16), jnp.float32),
          pltpu.SemaphoreType.DMA,
      ],
  )
  def degrees(sidx_hbm, ones_hbm, zeros_hbm, deg_hbm, acc, si_b, ones_v, ssem):
    cid = lax.axis_index("c")
    sid = lax.axis_index("s")
    pltpu.sync_copy(zeros_hbm, acc.at[pl.ds(sid * ZPT, ZPT)])
    pltpu.sync_copy(ones_hbm, ones_v)
    plsc.subcore_barrier()

    def block(k, carry):
      pltpu.sync_copy(sidx_hbm.at[cid, sid, k], si_b)
      for q in range(CB):
        pltpu.async_copy(ones_v, acc.at[si_b.at[q]], ssem, add=True).wait()
      return carry

    lax.fori_loop(0, NBLK, block, 0)
    plsc.subcore_barrier()
    pltpu.sync_copy(acc.at[pl.ds(sid * WPT, WPT)],
                    deg_hbm.at[cid, pl.ds(sid * OPT, WPT)])

  return degrees


def _layernorm(z, g, b):
  m = jnp.mean(z, axis=1, keepdims=True)
  v = jnp.mean((z - m) ** 2, axis=1, keepdims=True)
  return (z - m) * lax.rsqrt(v + 1e-5) * g + b


def _deg_inv_sqrt(deg_ref):
  deg = deg_ref[...][:, 0:1]
  return jnp.where(deg > 0, lax.rsqrt(deg), 0.0)


def _enc_x_body(xt, tsb, degx, w1a, w1b, b1, g1, bb1, fr2,
                zx0_o, zx1_o, p0_o, p1_o):
  ang = tsb[...] * fr2[...]
  lane = lax.broadcasted_iota(jnp.int32, ang.shape, 1)
  te = jnp.where(lane < 64, jnp.cos(ang), jnp.sin(ang))
  z = jnp.dot(xt[...], w1a[...], preferred_element_type=jnp.float32)
  z = z + jnp.dot(te, w1b[...], preferred_element_type=jnp.float32)
  z = _layernorm(z + b1[...], g1[...], bb1[...])
  z = jnp.maximum(z, 0.0)
  p = z * _deg_inv_sqrt(degx)
  zx0_o[...] = z[:, :128]
  zx1_o[...] = z[:, 128:]
  p0_o[...] = p[:, :128]
  p1_o[...] = p[:, 128:]


def _enc_y_body(yy, degy, w2, b2, g2, bb2, q0_o, q1_o):
  z = jnp.dot(yy[...], w2[...], preferred_element_type=jnp.float32)
  z = _layernorm(z + b2[...], g2[...], bb2[...])
  z = jnp.maximum(z, 0.0)
  q = z * _deg_inv_sqrt(degy)
  q0_o[...] = q[:, :128]
  q1_o[...] = q[:, 128:]


def _scale_body(p0, p1, s0, s1, deg_r, o0, o1):
  deg = deg_r[...][:, 0:1]
  d2 = jnp.where(deg > 0, 1.0 / deg, 0.0)
  o0[...] = p0[...] + d2 * s0[...]
  o1[...] = p1[...] + d2 * s1[...]


def _dec_body(zx0, zx1, v0, v1, degx, w3a, w3b, b3, g3, bb3, out_o):
  dx = _deg_inv_sqrt(degx)
  h0 = (zx0[...] + dx * v0[...]) * 0.25
  h1 = (zx1[...] + dx * v1[...]) * 0.25
  o = jnp.dot(h0, w3a[...], preferred_element_type=jnp.float32)
  o = o + jnp.dot(h1, w3b[...], preferred_element_type=jnp.float32)
  o = _layernorm(o + b3[...], g3[...], bb3[...])
  out_o[...] = jnp.tanh(o)


def _row_spec(r, cols):
  return pl.BlockSpec((r, cols), lambda i: (i, 0))


def _full_spec(shape):
  return pl.BlockSpec(shape, lambda i: tuple(0 for _ in shape))


def kernel(x_t, timestep, edge_index, y, W1, b1, ln1_g, ln1_b,
           W2, b2, ln2_g, ln2_b, W3, b3, ln3_g, ln3_b):
  N, D = x_t.shape
  M = y.shape[0]
  E = edge_index.shape[1]
  H = W1.shape[0]
  O = W3.shape[0]
  ept = E // NS

  f32 = jnp.float32

  def pack_idx(ei, padval):
    t = ei.astype(jnp.int32).reshape(NS, ept)
    t = jnp.pad(t, ((0, 0), (0, EPTP - ept)), constant_values=padval)
    return t.reshape(NS, NBLK, CB, CW)

  e0 = edge_index[0]
  e1 = edge_index[1]
  e0g = pack_idx(e0, 0)
  e0s = pack_idx(e0, SPARE)
  e1g = pack_idx(e1, 0)
  e1s = pack_idx(e1, SPARE)
  ge0 = jnp.stack([e0g, e0g + N])
  ge1 = jnp.stack([e1g, e1g + M])
  se0 = jnp.stack([e0s, e0s])
  se1 = jnp.stack([e1s, e1s])
  sidx2 = jnp.stack([e0s, e1s])

  def unpack(o):
    cols = o.shape[-1]
    return o.reshape(NS, OPT, cols)[:, :WPT].reshape(NS * WPT, cols)[:N]

  half = D // 2
  freqs = jnp.exp(-math.log(10000.0) *
                  jnp.arange(0, half, dtype=f32) / half)
  fr2 = jnp.concatenate([freqs, freqs]).reshape(1, D)
  tsb = jnp.broadcast_to(timestep.astype(f32)[:, None], (N, D))

  zerosb = jnp.zeros((ZPT, 128), f32)
  zeros16 = jnp.zeros((ZPT, 16), f32)
  ones16 = jnp.ones((CW, 16), f32)

  degrees = _make_degrees()
  spmm_p = _make_spmm()

  def spmm(a0, a1, gi, si):
    o = spmm_p(jnp.concatenate([a0, a1], axis=0), gi, si, zerosb)
    return unpack(o[0]), unpack(o[1])

  degf = degrees(sidx2, ones16, zeros16)
  degx = unpack(degf[0])
  degy = unpack(degf[1])

  R = 1000
  grid = (N // R,)
  w1aT = W1[:, :D].T
  w1bT = W1[:, D:].T
  zx0, zx1, p0, p1 = pl.pallas_call(
      _enc_x_body,
      grid=grid,
      in_specs=[
          _row_spec(R, D), _row_spec(R, D), _row_spec(R, 16),
          _full_spec((D, H)), _full_spec((D, H)), _full_spec((1, H)),
          _full_spec((1, H)), _full_spec((1, H)), _full_spec((1, D)),
      ],
      out_specs=[_row_spec(R, 128)] * 4,
      out_shape=[jax.ShapeDtypeStruct((N, 128), f32)] * 4,
  )(x_t, tsb, degx, w1aT, w1bT, b1.reshape(1, H), ln1_g.reshape(1, H),
    ln1_b.reshape(1, H), fr2)

  q0, q1 = pl.pallas_call(
      _enc_y_body,
      grid=grid,
      in_specs=[
          _row_spec(R, D), _row_spec(R, 16),
          _full_spec((D, H)), _full_spec((1, H)), _full_spec((1, H)),
          _full_spec((1, H)),
      ],
      out_specs=[_row_spec(R, 128)] * 2,
      out_shape=[jax.ShapeDtypeStruct((M, 128), f32)] * 2,
  )(y, degy, W2.T, b2.reshape(1, H), ln2_g.reshape(1, H),
    ln2_b.reshape(1, H))

  def scale(a0, a1, s0, s1, degs):
    return pl.pallas_call(
        _scale_body,
        grid=grid,
        in_specs=[_row_spec(R, 128)] * 4 + [_row_spec(R, 16)],
        out_specs=[_row_spec(R, 128)] * 2,
        out_shape=[jax.ShapeDtypeStruct((N, 128), f32)] * 2,
    )(a0, a1, s0, s1, degs)

  s0, s1 = spmm(q0, q1, ge1, se0)
  c0, c1 = scale(p0, p1, s0, s1, degx)
  u0, u1 = spmm(c0, c1, ge0, se1)
  d0, d1 = scale(q0, q1, u0, u1, degy)
  v0, v1 = spmm(d0, d1, ge1, se0)

  w3T = W3.T
  out = pl.pallas_call(
      _dec_body,
      grid=grid,
      in_specs=[_row_spec(R, 128)] * 4 + [
          _row_spec(R, 16),
          _full_spec((D, O)), _full_spec((D, O)), _full_spec((1, O)),
          _full_spec((1, O)), _full_spec((1, O)),
      ],
      out_specs=_row_spec(R, O),
      out_shape=jax.ShapeDtypeStruct((N, O), f32),
  )(zx0, zx1, v0, v1, degx, w3T[:D], w3T[D:], b3.reshape(1, O),
    ln3_g.reshape(1, O), ln3_b.reshape(1, O))
  return out
